# baseline (speedup 1.0000x reference)
import functools

import jax
import jax.numpy as jnp
from jax import lax
from jax.experimental import pallas as pl
from jax.experimental.pallas import tpu as pltpu
from jax.experimental.pallas import tpu_sc as plsc

N_LANES = 16
N_CORES = 2
N_SUBCORES = 16
GROUP = 6


def _gather_add_body(chunk, num_chunks,
                     range_hbm, params_hbm, idx_hbm, out_hbm,
                     table_sh, table_v,
                     idx_s0, idx_s1, idx_s2, rng_s0, rng_s1, rng_s2,
                     out_s0, out_s1,
                     idx_v0, idx_v1, rng_v0, rng_v1,
                     isem0, isem1, osem,
                     dsem_i0, dsem_i1, dsem_i2, dsem_o0, dsem_o1, tsem):
    s_id = lax.axis_index("s")
    c_id = lax.axis_index("c")

    idx_s = (idx_s0, idx_s1, idx_s2)
    rng_s = (rng_s0, rng_s1, rng_s2)
    out_s = (out_s0, out_s1)
    idx_v = (idx_v0, idx_v1)
    rng_v = (rng_v0, rng_v1)
    isems = (isem0, isem1)
    dsems_i = (dsem_i0, dsem_i1, dsem_i2)
    dsems_o = (dsem_o0, dsem_o1)
    tslice = pl.ds(s_id * chunk, chunk)

    @pl.when(s_id == 0)
    def _():
        pltpu.sync_copy(params_hbm, table_sh)

    plsc.subcore_barrier()
    pltpu.sync_copy(table_sh, table_v)

    def iter_body(jv, m, w_din, w_dout, do_din, do_sin, do_comp, do_dout):
        if w_din or w_dout:
            @pl.when(s_id == 0)
            def _():
                if w_din:
                    m1 = (m - 1) % 3
                    pltpu.make_async_copy(idx_hbm.at[0], idx_s[m1],
                                          dsems_i[m1]).wait()
                    pltpu.make_async_copy(range_hbm.at[0], rng_s[m1],
                                          dsems_i[m1]).wait()
                if w_dout:
                    m4 = (m - 4) % 2
                    pltpu.make_async_copy(out_s[m4], out_hbm.at[0],
                                          dsems_o[m4]).wait()

        plsc.subcore_barrier()

        if do_din:
            @pl.when(s_id == 0)
            def _():
                g = c_id * num_chunks + jv
                pltpu.async_copy(idx_hbm.at[g], idx_s[m % 3], dsems_i[m % 3])
                pltpu.async_copy(range_hbm.at[g], rng_s[m % 3], dsems_i[m % 3])

        if do_sin:
            m1 = (m - 1) % 3
            b1 = (m - 1) % 2
            pltpu.async_copy(idx_s[m1].at[tslice], idx_v[b1], isems[b1])
            pltpu.async_copy(rng_s[m1].at[tslice], rng_v[b1], isems[b1])

        if do_comp:
            m2 = (m - 2) % 3
            b2 = (m - 2) % 2
            pltpu.make_async_copy(idx_s[m2].at[tslice], idx_v[b2],
                                  isems[b2]).wait()
            pltpu.make_async_copy(rng_s[m2].at[tslice], rng_v[b2],
                                  isems[b2]).wait()
            idx_b = idx_v[b2]
            rng_b = rng_v[b2]

            @plsc.parallel_loop(0, chunk, step=N_LANES, unroll=8)
            def _(i):
                s = pl.ds(i, N_LANES)
                vals = plsc.load_gather(table_v, [idx_b[s]])
                rng_b[s] = rng_b[s] + vals

            pltpu.async_copy(rng_b, out_s[b2].at[tslice], osem).wait()

        plsc.subcore_barrier()

        if do_dout:
            @pl.when(s_id == 0)
            def _():
                m2 = (m - 2) % 2
                g = c_id * num_chunks + (jv - 2)
                pltpu.async_copy(out_s[m2], out_hbm.at[g], dsems_o[m2])

    total_iters = num_chunks + 4
    n_groups = (total_iters - 2 * GROUP) // GROUP
    assert total_iters == (n_groups + 2) * GROUP

    for j in range(GROUP):
        iter_body(j, j % GROUP,
                  w_din=j >= 1, w_dout=j >= 4,
                  do_din=j < num_chunks, do_sin=j >= 1,
                  do_comp=j >= 2, do_dout=j >= 2)

    def group_body(g, carry):
        j0 = GROUP + g * GROUP
        for jj in range(GROUP):
            iter_body(j0 + jj, jj,
                      w_din=True, w_dout=True,
                      do_din=True, do_sin=True,
                      do_comp=True, do_dout=True)
        return carry

    lax.fori_loop(0, n_groups, group_body, 0)

    for j in range(total_iters - GROUP, total_iters):
        iter_body(j, j % GROUP,
                  w_din=j - 1 < num_chunks, w_dout=True,
                  do_din=j < num_chunks, do_sin=j - 1 < num_chunks,
                  do_comp=j - 2 < num_chunks, do_dout=j - 2 < num_chunks)


@jax.jit
def _radar_bias_add(range_km, sensor_params, contact_indices):
    n_meas = range_km.shape[0]
    n_passes = sensor_params.shape[0]
    assert n_meas % N_CORES == 0
    half = n_meas // N_CORES
    chunk = 2048
    sc_chunk = N_SUBCORES * chunk
    assert half % sc_chunk == 0
    num_chunks = half // sc_chunk

    total_chunks = N_CORES * num_chunks
    mesh = plsc.VectorSubcoreMesh(core_axis_name="c", subcore_axis_name="s")
    body = functools.partial(_gather_add_body, chunk, num_chunks)
    f = pl.kernel(
        body,
        out_type=jax.ShapeDtypeStruct((total_chunks, sc_chunk), jnp.float32),
        mesh=mesh,
        compiler_params=pltpu.CompilerParams(needs_layout_passes=False),
        scratch_types=[
            pltpu.VMEM_SHARED((n_passes,), jnp.float32),
            pltpu.VMEM((n_passes,), jnp.float32),
            pltpu.VMEM_SHARED((sc_chunk,), jnp.int32),
            pltpu.VMEM_SHARED((sc_chunk,), jnp.int32),
            pltpu.VMEM_SHARED((sc_chunk,), jnp.int32),
            pltpu.VMEM_SHARED((sc_chunk,), jnp.float32),
            pltpu.VMEM_SHARED((sc_chunk,), jnp.float32),
            pltpu.VMEM_SHARED((sc_chunk,), jnp.float32),
            pltpu.VMEM_SHARED((sc_chunk,), jnp.float32),
            pltpu.VMEM_SHARED((sc_chunk,), jnp.float32),
            pltpu.VMEM((chunk,), jnp.int32),
            pltpu.VMEM((chunk,), jnp.int32),
            pltpu.VMEM((chunk,), jnp.float32),
            pltpu.VMEM((chunk,), jnp.float32),
            pltpu.SemaphoreType.DMA,
            pltpu.SemaphoreType.DMA,
            pltpu.SemaphoreType.DMA,
            pltpu.SemaphoreType.DMA,
            pltpu.SemaphoreType.DMA,
            pltpu.SemaphoreType.DMA,
            pltpu.SemaphoreType.DMA,
            pltpu.SemaphoreType.DMA,
            pltpu.SemaphoreType.DMA,
        ],
    )
    rng2 = range_km.reshape(total_chunks, sc_chunk)
    idx2 = contact_indices.reshape(total_chunks, sc_chunk)
    out = f(rng2, sensor_params, idx2)
    return out.reshape(n_meas)


def kernel(range_km, sensor_params, contact_indices):
    idx = contact_indices.astype(jnp.int32)
    return _radar_bias_add(range_km, sensor_params, idx)


# speedup vs baseline: 1.1196x; 1.1196x over previous
import functools

import jax
import jax.numpy as jnp
from jax import lax
from jax.experimental import pallas as pl
from jax.experimental.pallas import tpu as pltpu
from jax.experimental.pallas import tpu_sc as plsc

N_LANES = 16
N_CORES = 2
N_SUBCORES = 16
GROUP = 2


def _gather_add_body(chunk, num_chunks,
                     range_hbm, params_hbm, idx_hbm, out_hbm,
                     table_sh, table_v,
                     idx_s0, idx_s1, rng_s0, rng_s1, out_s0, out_s1,
                     idx_v0, idx_v1, rng_v0, rng_v1,
                     isem0, isem1, osem,
                     dsem_i0, dsem_i1, dsem_o0, dsem_o1):
    s_id = lax.axis_index("s")
    c_id = lax.axis_index("c")
    tile = c_id * N_SUBCORES + s_id

    idx_s = (idx_s0, idx_s1)
    rng_s = (rng_s0, rng_s1)
    out_s = (out_s0, out_s1)
    idx_v = (idx_v0, idx_v1)
    rng_v = (rng_v0, rng_v1)
    isems = (isem0, isem1)
    dsems_i = (dsem_i0, dsem_i1)
    dsems_o = (dsem_o0, dsem_o1)

    @pl.when(s_id == 0)
    def _():
        pltpu.sync_copy(params_hbm, table_sh)

    plsc.subcore_barrier()
    pltpu.sync_copy(table_sh, table_v)

    def iter_body(jv, m, w_sin, w_din, i_sin, i_din, comp, w_dout):
        if w_sin:
            b = m % 2
            pltpu.make_async_copy(idx_s[b].at[s_id], idx_v[b],
                                  isems[b]).wait()
            pltpu.make_async_copy(rng_s[b].at[s_id], rng_v[b],
                                  isems[b]).wait()

        if w_din:
            b1 = (m - 1) % 2
            pltpu.make_async_copy(idx_hbm.at[0], idx_s[b1].at[s_id],
                                  dsems_i[b1]).wait()
            pltpu.make_async_copy(range_hbm.at[0], rng_s[b1].at[s_id],
                                  dsems_i[b1]).wait()

        if i_sin:
            b1 = (m - 1) % 2
            pltpu.async_copy(idx_s[b1].at[s_id], idx_v[b1], isems[b1])
            pltpu.async_copy(rng_s[b1].at[s_id], rng_v[b1], isems[b1])

        if i_din:
            b = m % 2
            r = tile * num_chunks + jv
            pltpu.async_copy(idx_hbm.at[r], idx_s[b].at[s_id], dsems_i[b])
            pltpu.async_copy(range_hbm.at[r], rng_s[b].at[s_id], dsems_i[b])

        if comp:
            b2 = (m - 2) % 2
            idx_b = idx_v[b2]
            rng_b = rng_v[b2]

            @plsc.parallel_loop(0, chunk, step=N_LANES, unroll=8)
            def _(i):
                s = pl.ds(i, N_LANES)
                vals = plsc.load_gather(table_v, [idx_b[s]])
                rng_b[s] = rng_b[s] + vals

        if w_dout:
            b4 = (m - 4) % 2
            pltpu.make_async_copy(out_s[b4].at[s_id], out_hbm.at[0],
                                  dsems_o[b4]).wait()

        if comp:
            b2 = (m - 2) % 2
            rng_b = rng_v[b2]
            pltpu.async_copy(rng_b, out_s[b2].at[s_id], osem).wait()
            r2 = tile * num_chunks + (jv - 2)
            pltpu.async_copy(out_s[b2].at[s_id], out_hbm.at[r2], dsems_o[b2])

    total_iters = num_chunks + 4
    n_groups = (total_iters - 4 - 4) // GROUP
    assert total_iters == 8 + n_groups * GROUP

    for j in range(4):
        iter_body(j, j % 2,
                  w_sin=j >= 2, w_din=j >= 1, i_sin=j >= 1,
                  i_din=j < num_chunks, comp=j >= 2, w_dout=False)

    def group_body(g, carry):
        j0 = 4 + g * GROUP
        for jj in range(GROUP):
            iter_body(j0 + jj, jj,
                      w_sin=True, w_din=True, i_sin=True,
                      i_din=True, comp=True, w_dout=True)
        return carry

    lax.fori_loop(0, n_groups, group_body, 0)

    for j in range(total_iters - 4, total_iters):
        iter_body(j, j % 2,
                  w_sin=j - 2 < num_chunks, w_din=j - 1 < num_chunks,
                  i_sin=j - 1 < num_chunks, i_din=j < num_chunks,
                  comp=j - 2 < num_chunks, w_dout=True)


@jax.jit
def _radar_bias_add(range_km, sensor_params, contact_indices):
    n_meas = range_km.shape[0]
    n_passes = sensor_params.shape[0]
    n_tiles = N_CORES * N_SUBCORES
    chunk = 2048
    assert n_meas % (n_tiles * chunk) == 0
    num_chunks = n_meas // (n_tiles * chunk)
    total_rows = n_tiles * num_chunks

    mesh = plsc.VectorSubcoreMesh(core_axis_name="c", subcore_axis_name="s")
    body = functools.partial(_gather_add_body, chunk, num_chunks)
    f = pl.kernel(
        body,
        out_type=jax.ShapeDtypeStruct((total_rows, chunk), jnp.float32),
        mesh=mesh,
        compiler_params=pltpu.CompilerParams(needs_layout_passes=False),
        scratch_types=[
            pltpu.VMEM_SHARED((n_passes,), jnp.float32),
            pltpu.VMEM((n_passes,), jnp.float32),
            pltpu.VMEM_SHARED((N_SUBCORES, chunk), jnp.int32),
            pltpu.VMEM_SHARED((N_SUBCORES, chunk), jnp.int32),
            pltpu.VMEM_SHARED((N_SUBCORES, chunk), jnp.float32),
            pltpu.VMEM_SHARED((N_SUBCORES, chunk), jnp.float32),
            pltpu.VMEM_SHARED((N_SUBCORES, chunk), jnp.float32),
            pltpu.VMEM_SHARED((N_SUBCORES, chunk), jnp.float32),
            pltpu.VMEM((chunk,), jnp.int32),
            pltpu.VMEM((chunk,), jnp.int32),
            pltpu.VMEM((chunk,), jnp.float32),
            pltpu.VMEM((chunk,), jnp.float32),
            pltpu.SemaphoreType.DMA,
            pltpu.SemaphoreType.DMA,
            pltpu.SemaphoreType.DMA,
            pltpu.SemaphoreType.DMA,
            pltpu.SemaphoreType.DMA,
            pltpu.SemaphoreType.DMA,
            pltpu.SemaphoreType.DMA,
        ],
    )
    rng2 = range_km.reshape(total_rows, chunk)
    idx2 = contact_indices.reshape(total_rows, chunk)
    out = f(rng2, sensor_params, idx2)
    return out.reshape(n_meas)


def kernel(range_km, sensor_params, contact_indices):
    idx = contact_indices.astype(jnp.int32)
    return _radar_bias_add(range_km, sensor_params, idx)


# speedup vs baseline: 2.5586x; 2.2852x over previous
import functools

import jax
import jax.numpy as jnp
from jax import lax
from jax.experimental import pallas as pl
from jax.experimental.pallas import tpu as pltpu
from jax.experimental.pallas import tpu_sc as plsc

N_LANES = 16
NUM_WORKERS = 32


def _gather_add_body(per_worker, chunk, num_chunks,
                     range_hbm, params_hbm, idx_hbm, out_hbm,
                     table_sh, table_v,
                     idx_v0, idx_v1, rng_v0, rng_v1, res_v0, res_v1,
                     isem0, isem1, osem0, osem1, tsem):
    s_id = lax.axis_index("s")
    wid = s_id * 2 + lax.axis_index("c")
    base = wid * per_worker
    idx_v = (idx_v0, idx_v1)
    rng_v = (rng_v0, rng_v1)
    res_v = (res_v0, res_v1)
    isems = (isem0, isem1)
    osems = (osem0, osem1)

    in_copies = {}
    out_copies = {}

    def issue_in(j):
        b = j & 1
        off = base + j * chunk
        in_copies[j] = (
            pltpu.async_copy(idx_hbm.at[pl.ds(off, chunk)],
                             idx_v[b], isems[b]),
            pltpu.async_copy(range_hbm.at[pl.ds(off, chunk)],
                             rng_v[b], isems[b]),
        )

    issue_in(0)

    @pl.when(s_id == 0)
    def _():
        pltpu.sync_copy(params_hbm, table_sh)

    plsc.subcore_barrier()
    tcopy = pltpu.async_copy(table_sh, table_v, tsem)

    for j in range(num_chunks):
        b = j & 1
        if j + 1 < num_chunks:
            issue_in(j + 1)
        ci, cr = in_copies.pop(j)
        ci.wait()
        cr.wait()
        if j == 0:
            tcopy.wait()
        if j >= 2:
            out_copies.pop(j - 2).wait()

        idx_b = idx_v[b]
        rng_b = rng_v[b]
        res_b = res_v[b]

        @plsc.parallel_loop(0, chunk, step=N_LANES, unroll=8)
        def _(i):
            s = pl.ds(i, N_LANES)
            vals = plsc.load_gather(table_v, [idx_b[s]])
            res_b[s] = rng_b[s] + vals

        out_copies[j] = pltpu.async_copy(
            res_v[b], out_hbm.at[pl.ds(base + j * chunk, chunk)], osems[b])

    for j in sorted(out_copies):
        out_copies[j].wait()


@jax.jit
def _radar_bias_add(range_km, sensor_params, contact_indices):
    n_meas = range_km.shape[0]
    n_passes = sensor_params.shape[0]
    assert n_meas % NUM_WORKERS == 0
    per_worker = n_meas // NUM_WORKERS
    chunk = 4096
    assert per_worker % chunk == 0
    num_chunks = per_worker // chunk

    mesh = plsc.VectorSubcoreMesh(core_axis_name="c", subcore_axis_name="s")
    body = functools.partial(_gather_add_body, per_worker, chunk, num_chunks)
    f = pl.kernel(
        body,
        out_type=jax.ShapeDtypeStruct((n_meas,), jnp.float32),
        mesh=mesh,
        compiler_params=pltpu.CompilerParams(needs_layout_passes=False),
        scratch_types=[
            pltpu.VMEM_SHARED((n_passes,), jnp.float32),
            pltpu.VMEM((n_passes,), jnp.float32),
            pltpu.VMEM((chunk,), jnp.int32),
            pltpu.VMEM((chunk,), jnp.int32),
            pltpu.VMEM((chunk,), jnp.float32),
            pltpu.VMEM((chunk,), jnp.float32),
            pltpu.VMEM((chunk,), jnp.float32),
            pltpu.VMEM((chunk,), jnp.float32),
            pltpu.SemaphoreType.DMA,
            pltpu.SemaphoreType.DMA,
            pltpu.SemaphoreType.DMA,
            pltpu.SemaphoreType.DMA,
            pltpu.SemaphoreType.DMA,
        ],
    )
    return f(range_km, sensor_params, contact_indices)


def kernel(range_km, sensor_params, contact_indices):
    idx = contact_indices.astype(jnp.int32)
    return _radar_bias_add(range_km, sensor_params, idx)
